# Initial kernel scaffold; baseline (speedup 1.0000x reference)
import jax
import jax.numpy as jnp
from jax.experimental import pallas as pl


def kernel(x, edge_index, edge_attr, emb_table, W1_00, W1_01, W2_00, W2_11, W2_01, W2_10, W3_00, W3_11, Hw1, Hw2):
    raise NotImplementedError("write your pallas kernel here")


# speedup vs baseline: 4.2401x; 4.2401x over previous
import functools

import jax
import jax.numpy as jnp
from jax import lax
from jax.experimental import pallas as pl
from jax.experimental.pallas import tpu as pltpu
from jax.experimental.pallas import tpu_sc as plsc

N = 50000
E = 800000
NPAD = 50176
EPAD = 802816
B = 128
NTILES = 32
EPT = EPAD // NTILES
CHUNKS = EPT // B
RPT = NPAD // 16
ZROWS = 196

_INV16 = 1.0 / 16.0
_INV32 = 1.0 / 32.0
_SQRT3 = 3.0 ** 0.5
_INV_SQRT3 = 1.0 / _SQRT3
_INV8 = 1.0 / 8.0


def _used_comps(terms):
    comps = set()
    for t in terms:
        if t[0] == "scale":
            comps.add(t[1])
        elif t[0] == "dot":
            comps.update((0, 1, 2))
    return sorted(comps)


def _make_agg(F, G, terms, direct):
    mesh = plsc.VectorSubcoreMesh(core_axis_name="c", subcore_axis_name="s")
    comps = _used_comps(terms)

    scratch = [
        pltpu.VMEM((B,), jnp.int32),
        pltpu.VMEM((B,), jnp.int32),
        pltpu.VMEM((B, F), jnp.float32),
        pltpu.VMEM((B, G), jnp.float32),
        pltpu.VMEM((3, B), jnp.float32),
        pltpu.VMEM((ZROWS, G), jnp.float32),
        pltpu.VMEM_SHARED((NPAD, G), jnp.float32),
        pltpu.SemaphoreType.DMA,
    ]

    @functools.partial(
        pl.kernel,
        mesh=mesh,
        out_type=jax.ShapeDtypeStruct((2, NPAD, G), jnp.float32),
        scratch_types=scratch,
        compiler_params=pltpu.CompilerParams(use_tc_tiling_on_sc=False),
    )
    def agg(feat, srch, dsth, w0h, w1h, w2h, out, srcv, dstv, rows, srows,
            wv, zb, acc, sem):
        c = lax.axis_index("c")
        s = lax.axis_index("s")
        wid = s * 2 + c
        base_r = s * RPT

        zeros16 = jnp.zeros((16,), jnp.float32)

        def zloop(i, carry):
            r = i // (G // 16)
            col = (i % (G // 16)) * 16
            zb[r, pl.ds(col, 16)] = zeros16
            return carry

        lax.fori_loop(0, ZROWS * (G // 16), zloop, 0)
        for j in range(16):
            pltpu.sync_copy(zb, acc.at[pl.ds(base_r + j * ZROWS, ZROWS)])
        plsc.subcore_barrier()

        iota16 = lax.iota(jnp.int32, 16)

        def chunk(i, carry):
            base = wid * EPT + i * B
            pltpu.sync_copy(srch.at[pl.ds(base, B)], srcv)
            pltpu.sync_copy(dsth.at[pl.ds(base, B)], dstv)
            for comp, wh in ((0, w0h), (1, w1h), (2, w2h)):
                if comp in comps:
                    pltpu.sync_copy(wh.at[pl.ds(base, B)], wv.at[comp])
            pltpu.async_copy(feat.at[srcv], rows, sem).wait()
            if direct:
                pltpu.sync_copy(rows, acc.at[dstv], add=True)
            else:
                def grp(g, inner):
                    g0_ = g * 16
                    wvec = {}
                    for comp in comps:
                        wvec[comp] = wv[comp, pl.ds(g0_, 16)]
                    for j in range(16):
                        b = g0_ + j
                        ws = {comp: wvec[comp][j] for comp in comps}
                        for t in terms:
                            if t[0] == "copy":
                                _, sc, dc = t
                                srows[b, pl.ds(dc, 16)] = rows[b, pl.ds(sc, 16)]
                            elif t[0] == "scale":
                                _, comp, sc, dc = t
                                srows[b, pl.ds(dc, 16)] = (
                                    rows[b, pl.ds(sc, 16)] * ws[comp])
                            else:
                                _, dc = t
                                v = (rows[b, pl.ds(0, 16)] * ws[0]
                                     + rows[b, pl.ds(16, 16)] * ws[1]
                                     + rows[b, pl.ds(32, 16)] * ws[2])
                                srows[b, pl.ds(dc, 16)] = v
                    return inner

                lax.fori_loop(0, B // 16, grp, 0)
                pltpu.sync_copy(srows, acc.at[dstv], add=True)
            return carry

        lax.fori_loop(0, CHUNKS, chunk, 0)
        plsc.subcore_barrier()
        pltpu.sync_copy(acc.at[pl.ds(base_r, RPT)],
                        out.at[c, pl.ds(base_r, RPT)])

    return agg


@functools.lru_cache(maxsize=None)
def _agg_fn(F, G, terms, direct):
    return _make_agg(F, G, terms, direct)


def _agg(feat, srch, dsth, w0, w1, w2, G, terms, direct=False):
    F = feat.shape[1]
    out = _agg_fn(F, G, tuple(terms), direct)(feat, srch, dsth, w0, w1, w2)
    return out[0], out[1]


_NBLK = 1024
_NGRID = NPAD // _NBLK
_EBLK = 4096
_EGRID = EPAD // _EBLK


def _rowspec(width, nb=_NBLK):
    return pl.BlockSpec((nb, width), lambda i: (i, 0))


def _fullspec(shape):
    return pl.BlockSpec(shape, lambda i: tuple(0 for _ in shape))


def _tc_embed(x2, emb):
    def body(x_ref, emb_ref, o_ref):
        xb = x_ref[...]
        acc = jnp.zeros((_NBLK, 16), jnp.float32)
        for k in range(8):
            acc = acc + jnp.where(xb == k, 1.0, 0.0) * emb_ref[k:k + 1, :]
        o_ref[...] = acc

    return pl.pallas_call(
        body,
        grid=(_NGRID,),
        in_specs=[_rowspec(1), _fullspec((8, 16))],
        out_specs=_rowspec(16),
        out_shape=jax.ShapeDtypeStruct((NPAD, 16), jnp.float32),
    )(x2, emb)


def _tc_sh(ax, ay, az):
    def body(ax_ref, ay_ref, az_ref, w0_ref, w1_ref, w2_ref):
        vx = ax_ref[...]
        vy = ay_ref[...]
        vz = az_ref[...]
        rn = _SQRT3 * lax.rsqrt(vx * vx + vy * vy + vz * vz)
        w0_ref[...] = vy * rn
        w1_ref[...] = vz * rn
        w2_ref[...] = vx * rn

    espec = pl.BlockSpec((_EBLK,), lambda i: (i,))
    return pl.pallas_call(
        body,
        grid=(_EGRID,),
        in_specs=[espec] * 3,
        out_specs=[espec] * 3,
        out_shape=[jax.ShapeDtypeStruct((EPAD,), jnp.float32)] * 3,
    )(ax, ay, az)


def _tc_layer1(pa0, pa1, pb0, pb1, W100, W101, W201):
    def body(a0_ref, a1_ref, b0_ref, b1_ref, w00_ref, w01_ref, w201_ref,
             h0a_ref, h0b_ref, g0_ref, h1_ref):
        A = a0_ref[...] + a1_ref[...]
        Bm = b0_ref[...] + b1_ref[...]
        A0 = A[:, :16]
        Bc = (A[:, 16:], Bm[:, :16], Bm[:, 16:])
        h0 = jnp.dot(A0, w00_ref[...], preferred_element_type=jnp.float32) * _INV16
        h0a_ref[...] = h0[:, :32]
        h0b_ref[...] = h0[:, 32:]
        g0_ref[...] = jnp.dot(h0, w201_ref[...], preferred_element_type=jnp.float32)
        cols = [jnp.dot(Bc[c], w01_ref[...], preferred_element_type=jnp.float32)
                * _INV16 for c in range(3)]
        h1_ref[...] = jnp.concatenate(cols, axis=1)

    return pl.pallas_call(
        body,
        grid=(_NGRID,),
        in_specs=[_rowspec(32), _rowspec(32), _rowspec(32), _rowspec(32),
                  _fullspec((16, 48)), _fullspec((16, 16)), _fullspec((48, 16))],
        out_specs=[_rowspec(32), _rowspec(16), _rowspec(16), _rowspec(48)],
        out_shape=[jax.ShapeDtypeStruct((NPAD, 32), jnp.float32),
                   jax.ShapeDtypeStruct((NPAD, 16), jnp.float32),
                   jax.ShapeDtypeStruct((NPAD, 16), jnp.float32),
                   jax.ShapeDtypeStruct((NPAD, 48), jnp.float32)],
    )(pa0, pa1, pb0, pb1, W100, W101, W201)


def _tc_layer2(pc0, pc1, pd0, pd1, pe0, pe1, pf0, pf1, pg0, pg1, ph0, ph1,
               W200, W211, W210):
    def body(c0, c1, d0, d1, e0r, e1r, f0, f1, g0r, g1r, h0r, h1r,
             w00_ref, w11_ref, w10_ref, h0a_ref, h0b_ref, h1_ref):
        C = c0[...] + c1[...]
        D = d0[...] + d1[...]
        Ev = e0r[...] + e1r[...]
        F_ = f0[...] + f1[...]
        G_ = g0r[...] + g1r[...]
        H = h0r[...] + h1r[...]
        S0h0 = jnp.concatenate([C, D], axis=1)
        dots = G_[:, :16] * _INV_SQRT3
        h0 = (jnp.dot(S0h0, w00_ref[...], preferred_element_type=jnp.float32)
              + jnp.dot(dots, w11_ref[...], preferred_element_type=jnp.float32)) * _INV32
        h0a_ref[...] = h0[:, :32]
        h0b_ref[...] = h0[:, 32:]
        sg = (Ev[:, :16], Ev[:, 16:], F_[:, :16])
        s0h1 = (G_[:, 16:], H[:, :16], H[:, 16:])
        cols = [(sg[c] + jnp.dot(s0h1[c], w10_ref[...],
                                 preferred_element_type=jnp.float32)) * _INV32
                for c in range(3)]
        h1_ref[...] = jnp.concatenate(cols, axis=1)

    return pl.pallas_call(
        body,
        grid=(_NGRID,),
        in_specs=[_rowspec(32), _rowspec(32), _rowspec(16), _rowspec(16),
                  _rowspec(32), _rowspec(32), _rowspec(16), _rowspec(16),
                  _rowspec(32), _rowspec(32), _rowspec(32), _rowspec(32),
                  _fullspec((48, 48)), _fullspec((16, 48)), _fullspec((16, 16))],
        out_specs=[_rowspec(32), _rowspec(16), _rowspec(48)],
        out_shape=[jax.ShapeDtypeStruct((NPAD, 32), jnp.float32),
                   jax.ShapeDtypeStruct((NPAD, 16), jnp.float32),
                   jax.ShapeDtypeStruct((NPAD, 48), jnp.float32)],
    )(pc0, pc1, pd0, pd1, pe0, pe1, pf0, pf1, pg0, pg1, ph0, ph1,
      W200, W211, W210)


def _tc_layer3(pi0, pi1, pj0, pj1, pk0, pk1, W300, W311, Hw1, Hw2):
    def body(i0, i1, j0, j1, k0, k1, w00_ref, w11_ref, hw1_ref, hw2_ref, o_ref):
        I = i0[...] + i1[...]
        J = j0[...] + j1[...]
        K = k0[...] + k1[...]
        S0h0 = jnp.concatenate([I, J], axis=1)
        dots = K[:, :16] * _INV_SQRT3
        hemb = (jnp.dot(S0h0, w00_ref[...], preferred_element_type=jnp.float32)
                + jnp.dot(dots, w11_ref[...], preferred_element_type=jnp.float32)) * _INV32
        z = jax.nn.silu(jnp.dot(hemb, hw1_ref[...],
                                preferred_element_type=jnp.float32) * _INV8)
        o_ref[...] = jnp.dot(z, hw2_ref[...],
                             preferred_element_type=jnp.float32) * _INV8

    return pl.pallas_call(
        body,
        grid=(_NGRID,),
        in_specs=[_rowspec(32), _rowspec(32), _rowspec(16), _rowspec(16),
                  _rowspec(16), _rowspec(16),
                  _fullspec((48, 64)), _fullspec((16, 64)),
                  _fullspec((64, 64)), _fullspec((64, 4))],
        out_specs=_rowspec(4),
        out_shape=jax.ShapeDtypeStruct((NPAD, 4), jnp.float32),
    )(pi0, pi1, pj0, pj1, pk0, pk1, W300, W311, Hw1, Hw2)


def kernel(x, edge_index, edge_attr, emb_table, W1_00, W1_01, W2_00, W2_11,
           W2_01, W2_10, W3_00, W3_11, Hw1, Hw2):
    npad_e = EPAD - E
    pad_idx = (N + (jnp.arange(npad_e, dtype=jnp.int32) % 64)).astype(jnp.int32)
    srch = jnp.concatenate([edge_index[0].astype(jnp.int32), pad_idx])
    dsth = jnp.concatenate([edge_index[1].astype(jnp.int32), pad_idx])
    ones_e = jnp.ones((npad_e,), jnp.float32)
    ax = jnp.concatenate([edge_attr[:, 0], ones_e])
    ay = jnp.concatenate([edge_attr[:, 1], ones_e])
    az = jnp.concatenate([edge_attr[:, 2], ones_e])
    x2 = jnp.concatenate([x.astype(jnp.int32),
                          jnp.full((NPAD - N,), 8, jnp.int32)]).reshape(NPAD, 1)

    e0 = _tc_embed(x2, emb_table)
    w0, w1, w2 = _tc_sh(ax, ay, az)

    pa0, pa1 = _agg(e0, srch, dsth, w0, w1, w2, 32,
                    [("copy", 0, 0), ("scale", 0, 0, 16)])
    pb0, pb1 = _agg(e0, srch, dsth, w0, w1, w2, 32,
                    [("scale", 1, 0, 0), ("scale", 2, 0, 16)])
    h0a, h0b, g0, h1 = _tc_layer1(pa0, pa1, pb0, pb1, W1_00, W1_01, W2_01)

    pc0, pc1 = _agg(h0a, srch, dsth, w0, w1, w2, 32, [], direct=True)
    pd0, pd1 = _agg(h0b, srch, dsth, w0, w1, w2, 16, [], direct=True)
    pe0, pe1 = _agg(g0, srch, dsth, w0, w1, w2, 32,
                    [("scale", 0, 0, 0), ("scale", 1, 0, 16)])
    pf0, pf1 = _agg(g0, srch, dsth, w0, w1, w2, 16, [("scale", 2, 0, 0)])
    pg0, pg1 = _agg(h1, srch, dsth, w0, w1, w2, 32,
                    [("dot", 0), ("copy", 0, 16)])
    ph0, ph1 = _agg(h1, srch, dsth, w0, w1, w2, 32,
                    [("copy", 16, 0), ("copy", 32, 16)])
    h0a2, h0b2, h12 = _tc_layer2(pc0, pc1, pd0, pd1, pe0, pe1, pf0, pf1,
                                 pg0, pg1, ph0, ph1, W2_00, W2_11, W2_10)

    pi0, pi1 = _agg(h0a2, srch, dsth, w0, w1, w2, 32, [], direct=True)
    pj0, pj1 = _agg(h0b2, srch, dsth, w0, w1, w2, 16, [], direct=True)
    pk0, pk1 = _agg(h12, srch, dsth, w0, w1, w2, 16, [("dot", 0)])
    out = _tc_layer3(pi0, pi1, pj0, pj1, pk0, pk1, W3_00, W3_11, Hw1, Hw2)

    return out[:N]


# speedup vs baseline: 9.3006x; 2.1935x over previous
import functools

import jax
import jax.numpy as jnp
from jax import lax
from jax.experimental import pallas as pl
from jax.experimental.pallas import tpu as pltpu
from jax.experimental.pallas import tpu_sc as plsc

N = 50000
E = 800000
NPAD = 50176
SB = 512
NTILES = 32
SCPT = 50
EPT = SCPT * SB
EPAD = NTILES * EPT
NSC = EPAD // SB
RPT = NPAD // 16
ZROWS = 196

_INV16 = 1.0 / 16.0
_INV32 = 1.0 / 32.0
_SQRT3 = 3.0 ** 0.5
_INV_SQRT3 = 1.0 / _SQRT3
_INV8 = 1.0 / 8.0


def _used_comps(terms):
    comps = set()
    for t in terms:
        if t[0] == "scale":
            comps.add(t[1])
        elif t[0] == "dot":
            comps.update((0, 1, 2))
    return sorted(comps)


def _make_agg(Fs, G, terms, direct, sb):
    mesh = plsc.VectorSubcoreMesh(core_axis_name="c", subcore_axis_name="s")
    comps = _used_comps(terms)
    K = sb // 128
    scpt = EPT // sb
    nf = len(Fs)

    scratch = [pltpu.VMEM((2, 2 * K, 128), jnp.int32)]
    if comps:
        scratch.append(pltpu.VMEM((2, 3, sb), jnp.float32))
    for F in Fs:
        scratch.append(pltpu.VMEM((2, sb, F), jnp.float32))
    if not direct:
        scratch.append(pltpu.VMEM((sb, G), jnp.float32))
    scratch += [
        pltpu.VMEM((98, G), jnp.float32),
        pltpu.VMEM_SHARED((NPAD, G), jnp.float32),
        pltpu.SemaphoreType.DMA,
        pltpu.SemaphoreType.DMA,
    ]

    @functools.partial(
        pl.kernel,
        mesh=mesh,
        out_type=jax.ShapeDtypeStruct((2, NPAD, G), jnp.float32),
        scratch_types=scratch,
        compiler_params=pltpu.CompilerParams(use_tc_tiling_on_sc=False),
    )
    def agg(*refs):
        feats = refs[:nf]
        sdh, wph = refs[nf], refs[nf + 1]
        out = refs[nf + 2]
        sc_refs = list(refs[nf + 3:])
        idxb = sc_refs.pop(0)
        wb = sc_refs.pop(0) if comps else None
        rows = [sc_refs.pop(0) for _ in range(nf)]
        srows = None if direct else sc_refs.pop(0)
        zb, acc, sem0, sem1 = sc_refs
        sems = (sem0, sem1)

        c = lax.axis_index("c")
        s = lax.axis_index("s")
        wid = s * 2 + c
        base_r = s * RPT

        zeros16 = jnp.zeros((16,), jnp.float32)

        def zloop(i, carry):
            r = i // (G // 16)
            col = (i % (G // 16)) * 16
            zb[r, pl.ds(col, 16)] = zeros16
            return carry

        lax.fori_loop(0, 98 * (G // 16), zloop, 0)
        for j in range(32):
            pltpu.sync_copy(zb, acc.at[pl.ds(base_r + j * 98, 98)])
        plsc.subcore_barrier()

        sc0 = wid * scpt

        def load_meta(i, slot):
            pltpu.sync_copy(sdh.at[sc0 + i], idxb.at[slot])
            if comps:
                pltpu.sync_copy(wph.at[sc0 + i], wb.at[slot])

        def fire(slot):
            for fi in range(nf):
                for j in range(K):
                    pltpu.async_copy(feats[fi].at[idxb.at[slot, j]],
                                     rows[fi].at[slot, pl.ds(j * 128, 128)],
                                     sems[slot])

        def drain(slot):
            for fi in range(nf):
                for j in range(K):
                    pltpu.make_async_copy(feats[fi].at[idxb.at[slot, j]],
                                          rows[fi].at[slot, pl.ds(j * 128, 128)],
                                          sems[slot]).wait()

        def compute(slot):
            def grp(g, inner):
                g0_ = g * 16
                wvec = {}
                for comp in comps:
                    wvec[comp] = wb[slot, comp, pl.ds(g0_, 16)]
                for j in range(16):
                    b = g0_ + j
                    ws = {comp: wvec[comp][j] for comp in comps}
                    for t in terms:
                        if t[0] == "copy":
                            _, fi, dc = t
                            srows[b, pl.ds(dc, 16)] = rows[fi][slot, b, pl.ds(0, 16)]
                        elif t[0] == "scale":
                            _, comp, fi, dc = t
                            srows[b, pl.ds(dc, 16)] = (
                                rows[fi][slot, b, pl.ds(0, 16)] * ws[comp])
                        else:
                            _, dc = t
                            v = (rows[0][slot, b, pl.ds(0, 16)] * ws[0]
                                 + rows[1][slot, b, pl.ds(0, 16)] * ws[1]
                                 + rows[2][slot, b, pl.ds(0, 16)] * ws[2])
                            srows[b, pl.ds(dc, 16)] = v
                return inner

            lax.fori_loop(0, sb // 16, grp, 0)

        def scatter(slot):
            for j in range(K):
                if direct:
                    src_ref = rows[0].at[slot, pl.ds(j * 128, 128)]
                else:
                    src_ref = srows.at[pl.ds(j * 128, 128)]
                pltpu.sync_copy(src_ref, acc.at[idxb.at[slot, K + j]], add=True)

        load_meta(0, 0)
        fire(0)

        def pair(t, carry):
            for p in (0, 1):
                i = t * 2 + p

                @pl.when(i + 1 < scpt)
                def _():
                    load_meta(i + 1, 1 - p)
                    fire(1 - p)

                drain(p)
                if not direct:
                    compute(p)
                scatter(p)
            return carry

        lax.fori_loop(0, scpt // 2, pair, 0)
        plsc.subcore_barrier()
        pltpu.sync_copy(acc.at[pl.ds(base_r, RPT)],
                        out.at[c, pl.ds(base_r, RPT)])

    return agg


@functools.lru_cache(maxsize=None)
def _agg_fn(Fs, G, terms, direct, sb):
    return _make_agg(Fs, G, terms, direct, sb)


def _agg(feats, sdh, wph, G, terms, direct=False, sb=512):
    if not isinstance(feats, (list, tuple)):
        feats = [feats]
    Fs = tuple(f.shape[1] for f in feats)
    out = _agg_fn(Fs, G, tuple(terms), direct, sb)(*feats, sdh, wph)
    return out[0], out[1]


_NBLK = 1024
_NGRID = NPAD // _NBLK
_EBLK = 4096
_EGRID = EPAD // _EBLK


def _rowspec(width, nb=_NBLK):
    return pl.BlockSpec((nb, width), lambda i: (i, 0))


def _fullspec(shape):
    return pl.BlockSpec(shape, lambda i: tuple(0 for _ in shape))


def _tc_embed(x2, emb):
    def body(x_ref, emb_ref, o_ref):
        xb = x_ref[...]
        acc = jnp.zeros((_NBLK, 16), jnp.float32)
        for k in range(8):
            acc = acc + jnp.where(xb == k, 1.0, 0.0) * emb_ref[k:k + 1, :]
        o_ref[...] = acc

    return pl.pallas_call(
        body,
        grid=(_NGRID,),
        in_specs=[_rowspec(1), _fullspec((8, 16))],
        out_specs=_rowspec(16),
        out_shape=jax.ShapeDtypeStruct((NPAD, 16), jnp.float32),
    )(x2, emb)


def _tc_sh(ax, ay, az):
    def body(ax_ref, ay_ref, az_ref, w0_ref, w1_ref, w2_ref):
        vx = ax_ref[...]
        vy = ay_ref[...]
        vz = az_ref[...]
        rn = _SQRT3 * lax.rsqrt(vx * vx + vy * vy + vz * vz)
        w0_ref[...] = vy * rn
        w1_ref[...] = vz * rn
        w2_ref[...] = vx * rn

    espec = pl.BlockSpec((_EBLK,), lambda i: (i,))
    return pl.pallas_call(
        body,
        grid=(_EGRID,),
        in_specs=[espec] * 3,
        out_specs=[espec] * 3,
        out_shape=[jax.ShapeDtypeStruct((EPAD,), jnp.float32)] * 3,
    )(ax, ay, az)


def _tc_layer1(pa0, pa1, pb0, pb1, W100, W101, W201):
    def body(a0_ref, a1_ref, b0_ref, b1_ref, w00_ref, w01_ref, w201_ref,
             h0a_ref, h0b_ref, g0_ref, h1a_ref, h1b_ref, h1c_ref):
        A = a0_ref[...] + a1_ref[...]
        Bm = b0_ref[...] + b1_ref[...]
        A0 = A[:, :16]
        Bc = (A[:, 16:], Bm[:, :16], Bm[:, 16:])
        h0 = jnp.dot(A0, w00_ref[...], preferred_element_type=jnp.float32) * _INV16
        h0a_ref[...] = h0[:, :32]
        h0b_ref[...] = h0[:, 32:]
        g0_ref[...] = jnp.dot(h0, w201_ref[...], preferred_element_type=jnp.float32)
        for c, ref in enumerate((h1a_ref, h1b_ref, h1c_ref)):
            ref[...] = jnp.dot(Bc[c], w01_ref[...],
                               preferred_element_type=jnp.float32) * _INV16

    return pl.pallas_call(
        body,
        grid=(_NGRID,),
        in_specs=[_rowspec(32), _rowspec(32), _rowspec(32), _rowspec(32),
                  _fullspec((16, 48)), _fullspec((16, 16)), _fullspec((48, 16))],
        out_specs=[_rowspec(32), _rowspec(16), _rowspec(16),
                   _rowspec(16), _rowspec(16), _rowspec(16)],
        out_shape=[jax.ShapeDtypeStruct((NPAD, 32), jnp.float32)]
        + [jax.ShapeDtypeStruct((NPAD, 16), jnp.float32)] * 5,
    )(pa0, pa1, pb0, pb1, W100, W101, W201)


def _tc_layer2(pc0, pc1, pd0, pd1, pe0, pe1, pf0, pf1, pgd0, pgd1,
               ph_parts, W200, W211, W210):
    def body(c0, c1, d0, d1, e0r, e1r, f0, f1, gd0, gd1,
             hA0, hA1, hB0, hB1, hC0, hC1,
             w00_ref, w11_ref, w10_ref,
             h0a_ref, h0b_ref, h1a_ref, h1b_ref, h1c_ref):
        C = c0[...] + c1[...]
        D = d0[...] + d1[...]
        Ev = e0r[...] + e1r[...]
        F_ = f0[...] + f1[...]
        dots = (gd0[...] + gd1[...]) * _INV_SQRT3
        s0h1 = (hA0[...] + hA1[...], hB0[...] + hB1[...], hC0[...] + hC1[...])
        S0h0 = jnp.concatenate([C, D], axis=1)
        h0 = (jnp.dot(S0h0, w00_ref[...], preferred_element_type=jnp.float32)
              + jnp.dot(dots, w11_ref[...], preferred_element_type=jnp.float32)) * _INV32
        h0a_ref[...] = h0[:, :32]
        h0b_ref[...] = h0[:, 32:]
        sg = (Ev[:, :16], Ev[:, 16:], F_[:, :16])
        for c, ref in enumerate((h1a_ref, h1b_ref, h1c_ref)):
            ref[...] = (sg[c] + jnp.dot(s0h1[c], w10_ref[...],
                                        preferred_element_type=jnp.float32)) * _INV32

    return pl.pallas_call(
        body,
        grid=(_NGRID,),
        in_specs=[_rowspec(32), _rowspec(32), _rowspec(16), _rowspec(16),
                  _rowspec(32), _rowspec(32), _rowspec(16), _rowspec(16),
                  _rowspec(16), _rowspec(16),
                  _rowspec(16), _rowspec(16), _rowspec(16), _rowspec(16),
                  _rowspec(16), _rowspec(16),
                  _fullspec((48, 48)), _fullspec((16, 48)), _fullspec((16, 16))],
        out_specs=[_rowspec(32), _rowspec(16),
                   _rowspec(16), _rowspec(16), _rowspec(16)],
        out_shape=[jax.ShapeDtypeStruct((NPAD, 32), jnp.float32)]
        + [jax.ShapeDtypeStruct((NPAD, 16), jnp.float32)] * 4,
    )(pc0, pc1, pd0, pd1, pe0, pe1, pf0, pf1, pgd0, pgd1,
      ph_parts[0][0], ph_parts[0][1], ph_parts[1][0], ph_parts[1][1],
      ph_parts[2][0], ph_parts[2][1], W200, W211, W210)


def _tc_layer3(pi0, pi1, pj0, pj1, pk0, pk1, W300, W311, Hw1, Hw2):
    def body(i0, i1, j0, j1, k0, k1, w00_ref, w11_ref, hw1_ref, hw2_ref, o_ref):
        I = i0[...] + i1[...]
        J = j0[...] + j1[...]
        K = k0[...] + k1[...]
        S0h0 = jnp.concatenate([I, J], axis=1)
        dots = K[:, :16] * _INV_SQRT3
        hemb = (jnp.dot(S0h0, w00_ref[...], preferred_element_type=jnp.float32)
                + jnp.dot(dots, w11_ref[...], preferred_element_type=jnp.float32)) * _INV32
        z = jax.nn.silu(jnp.dot(hemb, hw1_ref[...],
                                preferred_element_type=jnp.float32) * _INV8)
        o_ref[...] = jnp.dot(z, hw2_ref[...],
                             preferred_element_type=jnp.float32) * _INV8

    return pl.pallas_call(
        body,
        grid=(_NGRID,),
        in_specs=[_rowspec(32), _rowspec(32), _rowspec(16), _rowspec(16),
                  _rowspec(16), _rowspec(16),
                  _fullspec((48, 64)), _fullspec((16, 64)),
                  _fullspec((64, 64)), _fullspec((64, 4))],
        out_specs=_rowspec(4),
        out_shape=jax.ShapeDtypeStruct((NPAD, 4), jnp.float32),
    )(pi0, pi1, pj0, pj1, pk0, pk1, W300, W311, Hw1, Hw2)


def kernel(x, edge_index, edge_attr, emb_table, W1_00, W1_01, W2_00, W2_11,
           W2_01, W2_10, W3_00, W3_11, Hw1, Hw2):
    npad_e = EPAD - E
    pad_idx = (N + (jnp.arange(npad_e, dtype=jnp.int32) % 160)).astype(jnp.int32)
    srch = jnp.concatenate([edge_index[0].astype(jnp.int32), pad_idx])
    dsth = jnp.concatenate([edge_index[1].astype(jnp.int32), pad_idx])

    def pack_idx(sb):
        k = sb // 128
        nsc = EPAD // sb
        return jnp.concatenate([srch.reshape(nsc, k, 128),
                                dsth.reshape(nsc, k, 128)], axis=1)

    sd256, sd512 = pack_idx(256), pack_idx(512)
    ones_e = jnp.ones((npad_e,), jnp.float32)
    ax = jnp.concatenate([edge_attr[:, 0], ones_e])
    ay = jnp.concatenate([edge_attr[:, 1], ones_e])
    az = jnp.concatenate([edge_attr[:, 2], ones_e])
    x2 = jnp.concatenate([x.astype(jnp.int32),
                          jnp.full((NPAD - N,), 8, jnp.int32)]).reshape(NPAD, 1)

    e0 = _tc_embed(x2, emb_table)
    w0, w1, w2 = _tc_sh(ax, ay, az)
    wcat = jnp.stack([w0, w1, w2], axis=0)
    wp256 = wcat.reshape(3, EPAD // 256, 256).transpose(1, 0, 2)
    wp512 = wcat.reshape(3, EPAD // 512, 512).transpose(1, 0, 2)

    pa0, pa1 = _agg(e0, sd256, wp256, 32,
                    [("copy", 0, 0), ("scale", 0, 0, 16)], sb=256)
    pb0, pb1 = _agg(e0, sd256, wp256, 32,
                    [("scale", 1, 0, 0), ("scale", 2, 0, 16)], sb=256)
    h0a, h0b, g0, h1a, h1b, h1c = _tc_layer1(pa0, pa1, pb0, pb1,
                                             W1_00, W1_01, W2_01)
    h1s = [h1a, h1b, h1c]

    pc0, pc1 = _agg(h0a, sd256, wp256, 32, [], direct=True, sb=256)
    pd0, pd1 = _agg(h0b, sd512, wp512, 16, [], direct=True, sb=512)
    pe0, pe1 = _agg(g0, sd256, wp256, 32,
                    [("scale", 0, 0, 0), ("scale", 1, 0, 16)], sb=256)
    pf0, pf1 = _agg(g0, sd512, wp512, 16, [("scale", 2, 0, 0)], sb=512)
    pgd0, pgd1 = _agg(h1s, sd512, wp512, 16, [("dot", 0)], sb=512)
    ph_parts = [_agg(h1s[cc], sd512, wp512, 16, [("copy", 0, 0)], sb=512)
                for cc in range(3)]
    h0a2, h0b2, h1a2, h1b2, h1c2 = _tc_layer2(
        pc0, pc1, pd0, pd1, pe0, pe1, pf0, pf1, pgd0, pgd1, ph_parts,
        W2_00, W2_11, W2_10)

    pi0, pi1 = _agg(h0a2, sd256, wp256, 32, [], direct=True, sb=256)
    pj0, pj1 = _agg(h0b2, sd512, wp512, 16, [], direct=True, sb=512)
    pk0, pk1 = _agg([h1a2, h1b2, h1c2], sd512, wp512, 16, [("dot", 0)], sb=512)
    out = _tc_layer3(pi0, pi1, pj0, pj1, pk0, pk1, W3_00, W3_11, Hw1, Hw2)

    return out[:N]


# speedup vs baseline: 10.9490x; 1.1772x over previous
import functools

import jax
import jax.numpy as jnp
from jax import lax
from jax.experimental import pallas as pl
from jax.experimental.pallas import tpu as pltpu
from jax.experimental.pallas import tpu_sc as plsc

N = 50000
E = 800000
NPAD = 50176
SB = 512
NTILES = 32
SCPT = 50
EPT = SCPT * SB
EPAD = NTILES * EPT
NSC = EPAD // SB
RPT = NPAD // 16
ZROWS = 196

_INV16 = 1.0 / 16.0
_INV32 = 1.0 / 32.0
_SQRT3 = 3.0 ** 0.5
_INV_SQRT3 = 1.0 / _SQRT3
_INV8 = 1.0 / 8.0


def _used_comps(terms):
    comps = set()
    for t in terms:
        if t[0] == "scale":
            comps.add(t[1])
        elif t[0] == "dot":
            comps.update((0, 1, 2))
    return sorted(comps)


_NR = 4


def _make_agg(Fs, G, terms, direct, sb):
    mesh = plsc.VectorSubcoreMesh(core_axis_name="c", subcore_axis_name="s")
    comps = _used_comps(terms)
    K = sb // 128
    scpt = EPT // sb
    assert scpt % _NR == 0
    nf = len(Fs)

    scratch = [pltpu.VMEM((_NR, 2 * K, 128), jnp.int32)]
    if comps:
        scratch.append(pltpu.VMEM((_NR, 3, sb), jnp.float32))
    for F in Fs:
        scratch.append(pltpu.VMEM((_NR, sb, F), jnp.float32))
    if not direct:
        scratch.append(pltpu.VMEM((_NR, sb, G), jnp.float32))
    scratch += [
        pltpu.VMEM((98, G), jnp.float32),
        pltpu.VMEM_SHARED((NPAD, G), jnp.float32),
    ]
    scratch += [pltpu.SemaphoreType.DMA] * (3 * _NR)

    @functools.partial(
        pl.kernel,
        mesh=mesh,
        out_type=jax.ShapeDtypeStruct((2, NPAD, G), jnp.float32),
        scratch_types=scratch,
        compiler_params=pltpu.CompilerParams(use_tc_tiling_on_sc=False),
    )
    def agg(*refs):
        feats = refs[:nf]
        sdh, wph = refs[nf], refs[nf + 1]
        out = refs[nf + 2]
        sc_refs = list(refs[nf + 3:])
        idxb = sc_refs.pop(0)
        wb = sc_refs.pop(0) if comps else None
        rows = [sc_refs.pop(0) for _ in range(nf)]
        srows = None if direct else sc_refs.pop(0)
        zb = sc_refs.pop(0)
        acc = sc_refs.pop(0)
        msem = sc_refs[0:_NR]
        gsem = sc_refs[_NR:2 * _NR]
        ssem = sc_refs[2 * _NR:3 * _NR]

        c = lax.axis_index("c")
        s = lax.axis_index("s")
        wid = s * 2 + c
        base_r = s * RPT

        zeros16 = jnp.zeros((16,), jnp.float32)

        def zloop(i, carry):
            r = i // (G // 16)
            col = (i % (G // 16)) * 16
            zb[r, pl.ds(col, 16)] = zeros16
            return carry

        lax.fori_loop(0, 98 * (G // 16), zloop, 0)
        for j in range(32):
            pltpu.sync_copy(zb, acc.at[pl.ds(base_r + j * 98, 98)])
        plsc.subcore_barrier()

        sc0 = wid * scpt

        def meta_start(i, m):
            pltpu.async_copy(sdh.at[sc0 + i], idxb.at[m], msem[m])
            if comps:
                pltpu.async_copy(wph.at[sc0 + i], wb.at[m], msem[m])

        def meta_wait(i, m):
            pltpu.make_async_copy(sdh.at[sc0 + i], idxb.at[m], msem[m]).wait()
            if comps:
                pltpu.make_async_copy(wph.at[sc0 + i], wb.at[m], msem[m]).wait()

        def fire(m):
            for fi in range(nf):
                for j in range(K):
                    pltpu.async_copy(feats[fi].at[idxb.at[m, j]],
                                     rows[fi].at[m, pl.ds(j * 128, 128)],
                                     gsem[m])

        def gdrain(m):
            for fi in range(nf):
                for j in range(K):
                    pltpu.make_async_copy(feats[fi].at[idxb.at[m, j]],
                                          rows[fi].at[m, pl.ds(j * 128, 128)],
                                          gsem[m]).wait()

        def compute(m):
            def grp(g, inner):
                g0_ = g * 16
                wvec = {}
                for comp in comps:
                    wvec[comp] = wb[m, comp, pl.ds(g0_, 16)]
                for j in range(16):
                    b = g0_ + j
                    ws = {comp: wvec[comp][j] for comp in comps}
                    for t in terms:
                        if t[0] == "copy":
                            _, fi, dc = t
                            srows[m, b, pl.ds(dc, 16)] = rows[fi][m, b, pl.ds(0, 16)]
                        elif t[0] == "scale":
                            _, comp, fi, dc = t
                            srows[m, b, pl.ds(dc, 16)] = (
                                rows[fi][m, b, pl.ds(0, 16)] * ws[comp])
                        else:
                            _, dc = t
                            v = (rows[0][m, b, pl.ds(0, 16)] * ws[0]
                                 + rows[1][m, b, pl.ds(0, 16)] * ws[1]
                                 + rows[2][m, b, pl.ds(0, 16)] * ws[2])
                            srows[m, b, pl.ds(dc, 16)] = v
                return inner

            lax.fori_loop(0, sb // 16, grp, 0)

        def _ssrc(m, j):
            if direct:
                return rows[0].at[m, pl.ds(j * 128, 128)]
            return srows.at[m, pl.ds(j * 128, 128)]

        def scat_start(m):
            for j in range(K):
                pltpu.async_copy(_ssrc(m, j), acc.at[idxb.at[m, K + j]],
                                 ssem[m], add=True)

        def scat_wait(m):
            for j in range(K):
                pltpu.make_async_copy(_ssrc(m, j), acc.at[idxb.at[m, K + j]],
                                      ssem[m]).wait()

        meta_start(0, 0)
        meta_start(1, 1)
        meta_wait(0, 0)
        fire(0)

        def quad(t, carry):
            for q in range(_NR):
                i = t * _NR + q

                @pl.when(i >= 2)
                def _():
                    scat_wait((q - 2) % _NR)

                @pl.when(i + 1 < scpt)
                def _():
                    meta_wait(i + 1, (q + 1) % _NR)
                    fire((q + 1) % _NR)

                @pl.when(i + 2 < scpt)
                def _():
                    meta_start(i + 2, (q + 2) % _NR)

                gdrain(q)
                if not direct:
                    compute(q)
                scat_start(q)
            return carry

        lax.fori_loop(0, scpt // _NR, quad, 0)
        scat_wait((scpt - 2) % _NR)
        scat_wait((scpt - 1) % _NR)
        plsc.subcore_barrier()
        pltpu.sync_copy(acc.at[pl.ds(base_r, RPT)],
                        out.at[c, pl.ds(base_r, RPT)])

    return agg


@functools.lru_cache(maxsize=None)
def _agg_fn(Fs, G, terms, direct, sb):
    return _make_agg(Fs, G, terms, direct, sb)


def _agg(feats, sdh, wph, G, terms, direct=False, sb=512):
    if not isinstance(feats, (list, tuple)):
        feats = [feats]
    Fs = tuple(f.shape[1] for f in feats)
    out = _agg_fn(Fs, G, tuple(terms), direct, sb)(*feats, sdh, wph)
    return out[0], out[1]


_NBLK = 1024
_NGRID = NPAD // _NBLK
_EBLK = 4096
_EGRID = EPAD // _EBLK


def _rowspec(width, nb=_NBLK):
    return pl.BlockSpec((nb, width), lambda i: (i, 0))


def _fullspec(shape):
    return pl.BlockSpec(shape, lambda i: tuple(0 for _ in shape))


def _tc_embed(x2, emb):
    def body(x_ref, emb_ref, o_ref):
        xb = x_ref[...]
        acc = jnp.zeros((_NBLK, 16), jnp.float32)
        for k in range(8):
            acc = acc + jnp.where(xb == k, 1.0, 0.0) * emb_ref[k:k + 1, :]
        o_ref[...] = acc

    return pl.pallas_call(
        body,
        grid=(_NGRID,),
        in_specs=[_rowspec(1), _fullspec((8, 16))],
        out_specs=_rowspec(16),
        out_shape=jax.ShapeDtypeStruct((NPAD, 16), jnp.float32),
    )(x2, emb)


def _tc_sh(ax, ay, az):
    def body(ax_ref, ay_ref, az_ref, w0_ref, w1_ref, w2_ref):
        vx = ax_ref[...]
        vy = ay_ref[...]
        vz = az_ref[...]
        rn = _SQRT3 * lax.rsqrt(vx * vx + vy * vy + vz * vz)
        w0_ref[...] = vy * rn
        w1_ref[...] = vz * rn
        w2_ref[...] = vx * rn

    espec = pl.BlockSpec((_EBLK,), lambda i: (i,))
    return pl.pallas_call(
        body,
        grid=(_EGRID,),
        in_specs=[espec] * 3,
        out_specs=[espec] * 3,
        out_shape=[jax.ShapeDtypeStruct((EPAD,), jnp.float32)] * 3,
    )(ax, ay, az)


def _tc_layer1(pa0, pa1, pb0, pb1, W100, W101, W201):
    def body(a0_ref, a1_ref, b0_ref, b1_ref, w00_ref, w01_ref, w201_ref,
             h0a_ref, h0b_ref, g0_ref, h1a_ref, h1b_ref, h1c_ref):
        A = a0_ref[...] + a1_ref[...]
        Bm = b0_ref[...] + b1_ref[...]
        A0 = A[:, :16]
        Bc = (A[:, 16:], Bm[:, :16], Bm[:, 16:])
        h0 = jnp.dot(A0, w00_ref[...], preferred_element_type=jnp.float32) * _INV16
        h0a_ref[...] = h0[:, :32]
        h0b_ref[...] = h0[:, 32:]
        g0_ref[...] = jnp.dot(h0, w201_ref[...], preferred_element_type=jnp.float32)
        for c, ref in enumerate((h1a_ref, h1b_ref, h1c_ref)):
            ref[...] = jnp.dot(Bc[c], w01_ref[...],
                               preferred_element_type=jnp.float32) * _INV16

    return pl.pallas_call(
        body,
        grid=(_NGRID,),
        in_specs=[_rowspec(32), _rowspec(32), _rowspec(32), _rowspec(32),
                  _fullspec((16, 48)), _fullspec((16, 16)), _fullspec((48, 16))],
        out_specs=[_rowspec(32), _rowspec(16), _rowspec(16),
                   _rowspec(16), _rowspec(16), _rowspec(16)],
        out_shape=[jax.ShapeDtypeStruct((NPAD, 32), jnp.float32)]
        + [jax.ShapeDtypeStruct((NPAD, 16), jnp.float32)] * 5,
    )(pa0, pa1, pb0, pb1, W100, W101, W201)


def _tc_layer2(pc0, pc1, pd0, pd1, pe0, pe1, pf0, pf1, pgd0, pgd1,
               ph_parts, W200, W211, W210):
    def body(c0, c1, d0, d1, e0r, e1r, f0, f1, gd0, gd1,
             hA0, hA1, hB0, hB1, hC0, hC1,
             w00_ref, w11_ref, w10_ref,
             h0a_ref, h0b_ref, h1a_ref, h1b_ref, h1c_ref):
        C = c0[...] + c1[...]
        D = d0[...] + d1[...]
        Ev = e0r[...] + e1r[...]
        F_ = f0[...] + f1[...]
        dots = (gd0[...] + gd1[...]) * _INV_SQRT3
        s0h1 = (hA0[...] + hA1[...], hB0[...] + hB1[...], hC0[...] + hC1[...])
        S0h0 = jnp.concatenate([C, D], axis=1)
        h0 = (jnp.dot(S0h0, w00_ref[...], preferred_element_type=jnp.float32)
              + jnp.dot(dots, w11_ref[...], preferred_element_type=jnp.float32)) * _INV32
        h0a_ref[...] = h0[:, :32]
        h0b_ref[...] = h0[:, 32:]
        sg = (Ev[:, :16], Ev[:, 16:], F_[:, :16])
        for c, ref in enumerate((h1a_ref, h1b_ref, h1c_ref)):
            ref[...] = (sg[c] + jnp.dot(s0h1[c], w10_ref[...],
                                        preferred_element_type=jnp.float32)) * _INV32

    return pl.pallas_call(
        body,
        grid=(_NGRID,),
        in_specs=[_rowspec(32), _rowspec(32), _rowspec(16), _rowspec(16),
                  _rowspec(32), _rowspec(32), _rowspec(16), _rowspec(16),
                  _rowspec(16), _rowspec(16),
                  _rowspec(16), _rowspec(16), _rowspec(16), _rowspec(16),
                  _rowspec(16), _rowspec(16),
                  _fullspec((48, 48)), _fullspec((16, 48)), _fullspec((16, 16))],
        out_specs=[_rowspec(32), _rowspec(16),
                   _rowspec(16), _rowspec(16), _rowspec(16)],
        out_shape=[jax.ShapeDtypeStruct((NPAD, 32), jnp.float32)]
        + [jax.ShapeDtypeStruct((NPAD, 16), jnp.float32)] * 4,
    )(pc0, pc1, pd0, pd1, pe0, pe1, pf0, pf1, pgd0, pgd1,
      ph_parts[0][0], ph_parts[0][1], ph_parts[1][0], ph_parts[1][1],
      ph_parts[2][0], ph_parts[2][1], W200, W211, W210)


def _tc_layer3(pi0, pi1, pj0, pj1, pk0, pk1, W300, W311, Hw1, Hw2):
    def body(i0, i1, j0, j1, k0, k1, w00_ref, w11_ref, hw1_ref, hw2_ref, o_ref):
        I = i0[...] + i1[...]
        J = j0[...] + j1[...]
        K = k0[...] + k1[...]
        S0h0 = jnp.concatenate([I, J], axis=1)
        dots = K[:, :16] * _INV_SQRT3
        hemb = (jnp.dot(S0h0, w00_ref[...], preferred_element_type=jnp.float32)
                + jnp.dot(dots, w11_ref[...], preferred_element_type=jnp.float32)) * _INV32
        z = jax.nn.silu(jnp.dot(hemb, hw1_ref[...],
                                preferred_element_type=jnp.float32) * _INV8)
        o_ref[...] = jnp.dot(z, hw2_ref[...],
                             preferred_element_type=jnp.float32) * _INV8

    return pl.pallas_call(
        body,
        grid=(_NGRID,),
        in_specs=[_rowspec(32), _rowspec(32), _rowspec(16), _rowspec(16),
                  _rowspec(16), _rowspec(16),
                  _fullspec((48, 64)), _fullspec((16, 64)),
                  _fullspec((64, 64)), _fullspec((64, 4))],
        out_specs=_rowspec(4),
        out_shape=jax.ShapeDtypeStruct((NPAD, 4), jnp.float32),
    )(pi0, pi1, pj0, pj1, pk0, pk1, W300, W311, Hw1, Hw2)


def kernel(x, edge_index, edge_attr, emb_table, W1_00, W1_01, W2_00, W2_11,
           W2_01, W2_10, W3_00, W3_11, Hw1, Hw2):
    npad_e = EPAD - E
    pad_idx = (N + (jnp.arange(npad_e, dtype=jnp.int32) % 160)).astype(jnp.int32)
    srch = jnp.concatenate([edge_index[0].astype(jnp.int32), pad_idx])
    dsth = jnp.concatenate([edge_index[1].astype(jnp.int32), pad_idx])

    def pack_idx(sb):
        k = sb // 128
        nsc = EPAD // sb
        return jnp.concatenate([srch.reshape(nsc, k, 128),
                                dsth.reshape(nsc, k, 128)], axis=1)

    sd128, sd256 = pack_idx(128), pack_idx(256)
    ones_e = jnp.ones((npad_e,), jnp.float32)
    ax = jnp.concatenate([edge_attr[:, 0], ones_e])
    ay = jnp.concatenate([edge_attr[:, 1], ones_e])
    az = jnp.concatenate([edge_attr[:, 2], ones_e])
    x2 = jnp.concatenate([x.astype(jnp.int32),
                          jnp.full((NPAD - N,), 8, jnp.int32)]).reshape(NPAD, 1)

    e0 = _tc_embed(x2, emb_table)
    w0, w1, w2 = _tc_sh(ax, ay, az)
    wcat = jnp.stack([w0, w1, w2], axis=0)
    wp128 = wcat.reshape(3, EPAD // 128, 128).transpose(1, 0, 2)
    wp256 = wcat.reshape(3, EPAD // 256, 256).transpose(1, 0, 2)

    pa0, pa1 = _agg(e0, sd128, wp128, 32,
                    [("copy", 0, 0), ("scale", 0, 0, 16)], sb=128)
    pb0, pb1 = _agg(e0, sd128, wp128, 32,
                    [("scale", 1, 0, 0), ("scale", 2, 0, 16)], sb=128)
    h0a, h0b, g0, h1a, h1b, h1c = _tc_layer1(pa0, pa1, pb0, pb1,
                                             W1_00, W1_01, W2_01)
    h1s = [h1a, h1b, h1c]

    pc0, pc1 = _agg(h0a, sd128, wp128, 32, [], direct=True, sb=128)
    pd0, pd1 = _agg(h0b, sd256, wp256, 16, [], direct=True, sb=256)
    pe0, pe1 = _agg(g0, sd128, wp128, 32,
                    [("scale", 0, 0, 0), ("scale", 1, 0, 16)], sb=128)
    pf0, pf1 = _agg(g0, sd256, wp256, 16, [("scale", 2, 0, 0)], sb=256)
    pgd0, pgd1 = _agg(h1s, sd256, wp256, 16, [("dot", 0)], sb=256)
    ph_parts = [_agg(h1s[cc], sd256, wp256, 16, [("copy", 0, 0)], sb=256)
                for cc in range(3)]
    h0a2, h0b2, h1a2, h1b2, h1c2 = _tc_layer2(
        pc0, pc1, pd0, pd1, pe0, pe1, pf0, pf1, pgd0, pgd1, ph_parts,
        W2_00, W2_11, W2_10)

    pi0, pi1 = _agg(h0a2, sd128, wp128, 32, [], direct=True, sb=128)
    pj0, pj1 = _agg(h0b2, sd256, wp256, 16, [], direct=True, sb=256)
    pk0, pk1 = _agg([h1a2, h1b2, h1c2], sd256, wp256, 16, [("dot", 0)], sb=256)
    out = _tc_layer3(pi0, pi1, pj0, pj1, pk0, pk1, W3_00, W3_11, Hw1, Hw2)

    return out[:N]


# speedup vs baseline: 11.1158x; 1.0152x over previous
import functools

import jax
import jax.numpy as jnp
from jax import lax
from jax.experimental import pallas as pl
from jax.experimental.pallas import tpu as pltpu
from jax.experimental.pallas import tpu_sc as plsc

N = 50000
E = 800000
NPAD = 50176
SB = 512
NTILES = 32
SCPT = 50
EPT = SCPT * SB
EPAD = NTILES * EPT
NSC = EPAD // SB
RPT = NPAD // 16
ZROWS = 196

_INV16 = 1.0 / 16.0
_INV32 = 1.0 / 32.0
_SQRT3 = 3.0 ** 0.5
_INV_SQRT3 = 1.0 / _SQRT3
_INV8 = 1.0 / 8.0


def _used_comps(terms):
    comps = set()
    for t in terms:
        if t[0] == "scale":
            comps.add(t[1])
        elif t[0] == "dot":
            comps.update((0, 1, 2))
    return sorted(comps)


_NR = 4


def _make_agg(Fs, G, terms, direct, sb):
    mesh = plsc.VectorSubcoreMesh(core_axis_name="c", subcore_axis_name="s")
    comps = _used_comps(terms)
    K = sb // 128
    scpt = EPT // sb
    assert scpt % _NR == 0
    nf = len(Fs)

    scratch = [pltpu.VMEM((_NR, 2 * K, 128), jnp.int32)]
    if comps:
        scratch.append(pltpu.VMEM((_NR, 3, sb), jnp.float32))
    for F in Fs:
        scratch.append(pltpu.VMEM((_NR, sb, F), jnp.float32))
    if not direct:
        scratch.append(pltpu.VMEM((_NR, sb, G), jnp.float32))
    scratch += [
        pltpu.VMEM((98, G), jnp.float32),
        pltpu.VMEM_SHARED((NPAD, G), jnp.float32),
    ]
    scratch += [pltpu.SemaphoreType.DMA] * (3 * _NR)

    @functools.partial(
        pl.kernel,
        mesh=mesh,
        out_type=jax.ShapeDtypeStruct((2, NPAD, G), jnp.float32),
        scratch_types=scratch,
        compiler_params=pltpu.CompilerParams(use_tc_tiling_on_sc=False),
    )
    def agg(*refs):
        feats = refs[:nf]
        sdh, wph = refs[nf], refs[nf + 1]
        out = refs[nf + 2]
        sc_refs = list(refs[nf + 3:])
        idxb = sc_refs.pop(0)
        wb = sc_refs.pop(0) if comps else None
        rows = [sc_refs.pop(0) for _ in range(nf)]
        srows = None if direct else sc_refs.pop(0)
        zb = sc_refs.pop(0)
        acc = sc_refs.pop(0)
        msem = sc_refs[0:_NR]
        gsem = sc_refs[_NR:2 * _NR]
        ssem = sc_refs[2 * _NR:3 * _NR]

        c = lax.axis_index("c")
        s = lax.axis_index("s")
        wid = s * 2 + c
        base_r = s * RPT

        zeros16 = jnp.zeros((16,), jnp.float32)

        def zloop(i, carry):
            r = i // (G // 16)
            col = (i % (G // 16)) * 16
            zb[r, pl.ds(col, 16)] = zeros16
            return carry

        lax.fori_loop(0, 98 * (G // 16), zloop, 0)
        for j in range(32):
            pltpu.sync_copy(zb, acc.at[pl.ds(base_r + j * 98, 98)])
        plsc.subcore_barrier()

        sc0 = wid * scpt

        def meta_start(i, m):
            pltpu.async_copy(sdh.at[sc0 + i], idxb.at[m], msem[m])
            if comps:
                pltpu.async_copy(wph.at[sc0 + i], wb.at[m], msem[m])

        def meta_wait(i, m):
            pltpu.make_async_copy(sdh.at[sc0 + i], idxb.at[m], msem[m]).wait()
            if comps:
                pltpu.make_async_copy(wph.at[sc0 + i], wb.at[m], msem[m]).wait()

        def fire(m):
            for fi in range(nf):
                for j in range(K):
                    pltpu.async_copy(feats[fi].at[idxb.at[m, j]],
                                     rows[fi].at[m, pl.ds(j * 128, 128)],
                                     gsem[m])

        def gdrain(m):
            for fi in range(nf):
                for j in range(K):
                    pltpu.make_async_copy(feats[fi].at[idxb.at[m, j]],
                                          rows[fi].at[m, pl.ds(j * 128, 128)],
                                          gsem[m]).wait()

        def compute(m):
            def grp(g, inner):
                g0_ = g * 16
                wvec = {}
                for comp in comps:
                    wvec[comp] = wb[m, comp, pl.ds(g0_, 16)]
                for j in range(16):
                    b = g0_ + j
                    ws = {comp: wvec[comp][j] for comp in comps}
                    for t in terms:
                        if t[0] == "copy":
                            _, fi, dc = t
                            srows[m, b, pl.ds(dc, 16)] = rows[fi][m, b, pl.ds(0, 16)]
                        elif t[0] == "scale":
                            _, comp, fi, dc = t
                            srows[m, b, pl.ds(dc, 16)] = (
                                rows[fi][m, b, pl.ds(0, 16)] * ws[comp])
                        else:
                            _, dc = t
                            v = (rows[0][m, b, pl.ds(0, 16)] * ws[0]
                                 + rows[1][m, b, pl.ds(0, 16)] * ws[1]
                                 + rows[2][m, b, pl.ds(0, 16)] * ws[2])
                            srows[m, b, pl.ds(dc, 16)] = v
                return inner

            lax.fori_loop(0, sb // 16, grp, 0)

        def _ssrc(m, j):
            if direct:
                return rows[0].at[m, pl.ds(j * 128, 128)]
            return srows.at[m, pl.ds(j * 128, 128)]

        def scat_start(m):
            for j in range(K):
                pltpu.async_copy(_ssrc(m, j), acc.at[idxb.at[m, K + j]],
                                 ssem[m], add=True)

        def scat_wait(m):
            for j in range(K):
                pltpu.make_async_copy(_ssrc(m, j), acc.at[idxb.at[m, K + j]],
                                      ssem[m]).wait()

        meta_start(0, 0)
        meta_start(1, 1)
        meta_wait(0, 0)
        fire(0)

        def quad(t, carry):
            for q in range(_NR):
                i = t * _NR + q

                @pl.when(i >= 2)
                def _():
                    scat_wait((q - 2) % _NR)

                @pl.when(i + 1 < scpt)
                def _():
                    meta_wait(i + 1, (q + 1) % _NR)
                    fire((q + 1) % _NR)

                @pl.when(i + 2 < scpt)
                def _():
                    meta_start(i + 2, (q + 2) % _NR)

                gdrain(q)
                if not direct:
                    compute(q)
                scat_start(q)
            return carry

        lax.fori_loop(0, scpt // _NR, quad, 0)
        scat_wait((scpt - 2) % _NR)
        scat_wait((scpt - 1) % _NR)
        plsc.subcore_barrier()
        pltpu.sync_copy(acc.at[pl.ds(base_r, RPT)],
                        out.at[c, pl.ds(base_r, RPT)])

    return agg


def _make_multi(G, sb, tasks0, tasks1, n_outs, nf_max):
    mesh = plsc.VectorSubcoreMesh(core_axis_name="c", subcore_axis_name="s")
    K = sb // 128
    scpt = (EPAD // 16) // sb
    assert scpt % _NR == 0
    all_tasks = list(tasks0) + list(tasks1)
    n_feat_args = sum(t[0] for t in all_tasks)
    any_comps = any(_used_comps(t[1]) for t in all_tasks)

    scratch = [pltpu.VMEM((_NR, 2 * K, 128), jnp.int32)]
    if any_comps:
        scratch.append(pltpu.VMEM((_NR, 3, sb), jnp.float32))
    for _ in range(nf_max):
        scratch.append(pltpu.VMEM((_NR, sb, 16), jnp.float32))
    scratch.append(pltpu.VMEM((_NR, sb, G), jnp.float32))
    scratch += [
        pltpu.VMEM((98, G), jnp.float32),
        pltpu.VMEM_SHARED((NPAD, G), jnp.float32),
    ]
    scratch += [pltpu.SemaphoreType.DMA] * (3 * _NR + 1)

    @functools.partial(
        pl.kernel,
        mesh=mesh,
        out_type=[jax.ShapeDtypeStruct((NPAD, G), jnp.float32)] * n_outs,
        scratch_types=scratch,
        compiler_params=pltpu.CompilerParams(use_tc_tiling_on_sc=False),
    )
    def agg(*refs):
        feat_args = refs[:n_feat_args]
        sdh, wph = refs[n_feat_args], refs[n_feat_args + 1]
        outs = refs[n_feat_args + 2:n_feat_args + 2 + n_outs]
        sc_refs = list(refs[n_feat_args + 2 + n_outs:])
        idxb = sc_refs.pop(0)
        wb = sc_refs.pop(0) if any_comps else None
        rows = [sc_refs.pop(0) for _ in range(nf_max)]
        srows = sc_refs.pop(0)
        zb = sc_refs.pop(0)
        acc = sc_refs.pop(0)
        msem = sc_refs[0:_NR]
        gsem = sc_refs[_NR:2 * _NR]
        ssem = sc_refs[2 * _NR:3 * _NR]
        zsem = sc_refs[3 * _NR]

        c = lax.axis_index("c")
        s = lax.axis_index("s")
        base_r = s * RPT
        sc0 = s * scpt

        zeros16 = jnp.zeros((16,), jnp.float32)

        def zloop(i, carry):
            r = i // (G // 16)
            col = (i % (G // 16)) * 16
            zb[r, pl.ds(col, 16)] = zeros16
            return carry

        lax.fori_loop(0, 98 * (G // 16), zloop, 0)
        for j in range(32):
            pltpu.sync_copy(zb, acc.at[pl.ds(base_r + j * 98, 98)])
        plsc.subcore_barrier()

        def run_task(feats, terms, direct):
            comps = _used_comps(terms)
            nf = len(feats)

            def meta_start(i, m):
                pltpu.async_copy(sdh.at[sc0 + i], idxb.at[m], msem[m])
                if comps:
                    pltpu.async_copy(wph.at[sc0 + i], wb.at[m], msem[m])

            def meta_wait(i, m):
                pltpu.make_async_copy(sdh.at[sc0 + i], idxb.at[m],
                                      msem[m]).wait()
                if comps:
                    pltpu.make_async_copy(wph.at[sc0 + i], wb.at[m],
                                          msem[m]).wait()

            def fire(m):
                for fi in range(nf):
                    for j in range(K):
                        pltpu.async_copy(feats[fi].at[idxb.at[m, j]],
                                         rows[fi].at[m, pl.ds(j * 128, 128)],
                                         gsem[m])

            def gdrain(m):
                for fi in range(nf):
                    for j in range(K):
                        pltpu.make_async_copy(
                            feats[fi].at[idxb.at[m, j]],
                            rows[fi].at[m, pl.ds(j * 128, 128)],
                            gsem[m]).wait()

            def compute(m):
                def grp(g, inner):
                    g0_ = g * 16
                    wvec = {}
                    for comp in comps:
                        wvec[comp] = wb[m, comp, pl.ds(g0_, 16)]
                    for j in range(16):
                        b = g0_ + j
                        ws = {comp: wvec[comp][j] for comp in comps}
                        for t in terms:
                            if t[0] == "copy":
                                _, fi, dc = t
                                srows[m, b, pl.ds(dc, 16)] = (
                                    rows[fi][m, b, pl.ds(0, 16)])
                            elif t[0] == "scale":
                                _, comp, fi, dc = t
                                srows[m, b, pl.ds(dc, 16)] = (
                                    rows[fi][m, b, pl.ds(0, 16)] * ws[comp])
                            else:
                                _, dc = t
                                v = (rows[0][m, b, pl.ds(0, 16)] * ws[0]
                                     + rows[1][m, b, pl.ds(0, 16)] * ws[1]
                                     + rows[2][m, b, pl.ds(0, 16)] * ws[2])
                                srows[m, b, pl.ds(dc, 16)] = v
                    return inner

                lax.fori_loop(0, sb // 16, grp, 0)

            def _ssrc(m, j):
                if direct:
                    return rows[0].at[m, pl.ds(j * 128, 128)]
                return srows.at[m, pl.ds(j * 128, 128)]

            def scat_start(m):
                for j in range(K):
                    pltpu.async_copy(_ssrc(m, j), acc.at[idxb.at[m, K + j]],
                                     ssem[m], add=True)

            def scat_wait(m):
                for j in range(K):
                    pltpu.make_async_copy(_ssrc(m, j),
                                          acc.at[idxb.at[m, K + j]],
                                          ssem[m]).wait()

            meta_start(0, 0)
            meta_start(1, 1)
            meta_wait(0, 0)
            fire(0)

            def quad(t, carry):
                for q in range(_NR):
                    i = t * _NR + q

                    @pl.when(i >= 2)
                    def _():
                        scat_wait((q - 2) % _NR)

                    @pl.when(i + 1 < scpt)
                    def _():
                        meta_wait(i + 1, (q + 1) % _NR)
                        fire((q + 1) % _NR)

                    @pl.when(i + 2 < scpt)
                    def _():
                        meta_start(i + 2, (q + 2) % _NR)

                    gdrain(q)
                    if not direct:
                        compute(q)
                    scat_start(q)
                return carry

            lax.fori_loop(0, scpt // _NR, quad, 0)
            scat_wait((scpt - 2) % _NR)
            scat_wait((scpt - 1) % _NR)

        assert len(tasks0) == 1 and len(tasks1) == 1
        nf0 = tasks0[0][0]
        feats0 = tuple(feat_args[:nf0])
        feats1 = tuple(feat_args[nf0:nf0 + tasks1[0][0]])
        out0 = outs[tasks0[0][3]]
        out1 = outs[tasks1[0][3]]

        @pl.when(c == 0)
        def _():
            run_task(feats0, tuple(tasks0[0][1]), tasks0[0][2])

        @pl.when(c == 1)
        def _():
            run_task(feats1, tuple(tasks1[0][1]), tasks1[0][2])

        plsc.subcore_barrier()

        @pl.when(c == 0)
        def _():
            pltpu.sync_copy(acc.at[pl.ds(base_r, RPT)],
                            out0.at[pl.ds(base_r, RPT)])

        @pl.when(c == 1)
        def _():
            pltpu.sync_copy(acc.at[pl.ds(base_r, RPT)],
                            out1.at[pl.ds(base_r, RPT)])

    return agg


@functools.lru_cache(maxsize=None)
def _multi_fn(G, sb, tasks0, tasks1, n_outs, nf_max):
    return _make_multi(G, sb, tasks0, tasks1, n_outs, nf_max)


@functools.lru_cache(maxsize=None)
def _agg_fn(Fs, G, terms, direct, sb):
    return _make_agg(Fs, G, terms, direct, sb)


def _agg(feats, sdh, wph, G, terms, direct=False, sb=512):
    if not isinstance(feats, (list, tuple)):
        feats = [feats]
    Fs = tuple(f.shape[1] for f in feats)
    out = _agg_fn(Fs, G, tuple(terms), direct, sb)(*feats, sdh, wph)
    return out[0], out[1]


_NBLK = 1024
_NGRID = NPAD // _NBLK
_EBLK = 4096
_EGRID = EPAD // _EBLK


def _rowspec(width, nb=_NBLK):
    return pl.BlockSpec((nb, width), lambda i: (i, 0))


def _fullspec(shape):
    return pl.BlockSpec(shape, lambda i: tuple(0 for _ in shape))


def _tc_embed(x2, emb):
    def body(x_ref, emb_ref, o_ref):
        xb = x_ref[...]
        acc = jnp.zeros((_NBLK, 16), jnp.float32)
        for k in range(8):
            acc = acc + jnp.where(xb == k, 1.0, 0.0) * emb_ref[k:k + 1, :]
        o_ref[...] = acc

    return pl.pallas_call(
        body,
        grid=(_NGRID,),
        in_specs=[_rowspec(1), _fullspec((8, 16))],
        out_specs=_rowspec(16),
        out_shape=jax.ShapeDtypeStruct((NPAD, 16), jnp.float32),
    )(x2, emb)


def _tc_sh(ax, ay, az):
    def body(ax_ref, ay_ref, az_ref, w0_ref, w1_ref, w2_ref):
        vx = ax_ref[...]
        vy = ay_ref[...]
        vz = az_ref[...]
        rn = _SQRT3 / jnp.sqrt(vx * vx + vy * vy + vz * vz)
        w0_ref[...] = vy * rn
        w1_ref[...] = vz * rn
        w2_ref[...] = vx * rn

    espec = pl.BlockSpec((_EBLK,), lambda i: (i,))
    return pl.pallas_call(
        body,
        grid=(_EGRID,),
        in_specs=[espec] * 3,
        out_specs=[espec] * 3,
        out_shape=[jax.ShapeDtypeStruct((EPAD,), jnp.float32)] * 3,
    )(ax, ay, az)


def _tc_layer1(pa, pb, W100, W101, W201):
    def body(a_ref, b_ref, w00_ref, w01_ref, w201_ref,
             h0a_ref, h0b_ref, g0_ref, h1a_ref, h1b_ref, h1c_ref):
        A = a_ref[...]
        Bm = b_ref[...]
        A0 = A[:, :16]
        Bc = (A[:, 16:], Bm[:, :16], Bm[:, 16:])
        h0 = jnp.dot(A0, w00_ref[...], preferred_element_type=jnp.float32, precision=lax.Precision.HIGHEST) * _INV16
        h0a_ref[...] = h0[:, :32]
        h0b_ref[...] = h0[:, 32:]
        g0_ref[...] = jnp.dot(h0, w201_ref[...], preferred_element_type=jnp.float32, precision=lax.Precision.HIGHEST)
        for c, ref in enumerate((h1a_ref, h1b_ref, h1c_ref)):
            ref[...] = jnp.dot(Bc[c], w01_ref[...],
                               preferred_element_type=jnp.float32, precision=lax.Precision.HIGHEST) * _INV16

    return pl.pallas_call(
        body,
        grid=(_NGRID,),
        in_specs=[_rowspec(32), _rowspec(32),
                  _fullspec((16, 48)), _fullspec((16, 16)), _fullspec((48, 16))],
        out_specs=[_rowspec(32), _rowspec(16), _rowspec(16),
                   _rowspec(16), _rowspec(16), _rowspec(16)],
        out_shape=[jax.ShapeDtypeStruct((NPAD, 32), jnp.float32)]
        + [jax.ShapeDtypeStruct((NPAD, 16), jnp.float32)] * 5,
    )(pa, pb, W100, W101, W201)


def _tc_layer2(pc0, pc1, pd, pe0, pe1, pf, pgd, phs, W200, W211, W210):
    def body(c0, c1, d0, e0r, e1r, f0, gd0, hA0, hB0, hC0,
             w00_ref, w11_ref, w10_ref,
             h0a_ref, h0b_ref, h1a_ref, h1b_ref, h1c_ref):
        C = c0[...] + c1[...]
        D = d0[...]
        Ev = e0r[...] + e1r[...]
        F_ = f0[...]
        dots = gd0[...] * _INV_SQRT3
        s0h1 = (hA0[...], hB0[...], hC0[...])
        S0h0 = jnp.concatenate([C, D], axis=1)
        h0 = (jnp.dot(S0h0, w00_ref[...], preferred_element_type=jnp.float32, precision=lax.Precision.HIGHEST)
              + jnp.dot(dots, w11_ref[...], preferred_element_type=jnp.float32, precision=lax.Precision.HIGHEST)) * _INV32
        h0a_ref[...] = h0[:, :32]
        h0b_ref[...] = h0[:, 32:]
        sg = (Ev[:, :16], Ev[:, 16:], F_[:, :16])
        for c, ref in enumerate((h1a_ref, h1b_ref, h1c_ref)):
            ref[...] = (sg[c] + jnp.dot(s0h1[c], w10_ref[...],
                                        preferred_element_type=jnp.float32, precision=lax.Precision.HIGHEST)) * _INV32

    return pl.pallas_call(
        body,
        grid=(_NGRID,),
        in_specs=[_rowspec(32), _rowspec(32), _rowspec(16),
                  _rowspec(32), _rowspec(32), _rowspec(16),
                  _rowspec(16), _rowspec(16), _rowspec(16), _rowspec(16),
                  _fullspec((48, 48)), _fullspec((16, 48)), _fullspec((16, 16))],
        out_specs=[_rowspec(32), _rowspec(16),
                   _rowspec(16), _rowspec(16), _rowspec(16)],
        out_shape=[jax.ShapeDtypeStruct((NPAD, 32), jnp.float32)]
        + [jax.ShapeDtypeStruct((NPAD, 16), jnp.float32)] * 4,
    )(pc0, pc1, pd, pe0, pe1, pf, pgd, phs[0], phs[1], phs[2],
      W200, W211, W210)


def _tc_layer3(pi0, pi1, pj, pk, W300, W311, Hw1, Hw2):
    def body(i0, i1, j0, k0, w00_ref, w11_ref, hw1_ref, hw2_ref, o_ref):
        I = i0[...] + i1[...]
        J = j0[...]
        K = k0[...]
        S0h0 = jnp.concatenate([I, J], axis=1)
        dots = K[:, :16] * _INV_SQRT3
        hemb = (jnp.dot(S0h0, w00_ref[...], preferred_element_type=jnp.float32, precision=lax.Precision.HIGHEST)
                + jnp.dot(dots, w11_ref[...], preferred_element_type=jnp.float32, precision=lax.Precision.HIGHEST)) * _INV32
        z = jax.nn.silu(jnp.dot(hemb, hw1_ref[...],
                                preferred_element_type=jnp.float32, precision=lax.Precision.HIGHEST) * _INV8)
        o_ref[...] = jnp.dot(z, hw2_ref[...],
                             preferred_element_type=jnp.float32, precision=lax.Precision.HIGHEST) * _INV8

    return pl.pallas_call(
        body,
        grid=(_NGRID,),
        in_specs=[_rowspec(32), _rowspec(32), _rowspec(16), _rowspec(16),
                  _fullspec((48, 64)), _fullspec((16, 64)),
                  _fullspec((64, 64)), _fullspec((64, 4))],
        out_specs=_rowspec(4),
        out_shape=jax.ShapeDtypeStruct((NPAD, 4), jnp.float32),
    )(pi0, pi1, pj, pk, W300, W311, Hw1, Hw2)


def kernel(x, edge_index, edge_attr, emb_table, W1_00, W1_01, W2_00, W2_11,
           W2_01, W2_10, W3_00, W3_11, Hw1, Hw2):
    npad_e = EPAD - E
    pad_idx = (N + (jnp.arange(npad_e, dtype=jnp.int32) % 160)).astype(jnp.int32)
    srch = jnp.concatenate([edge_index[0].astype(jnp.int32), pad_idx])
    dsth = jnp.concatenate([edge_index[1].astype(jnp.int32), pad_idx])

    def pack_idx(sb):
        k = sb // 128
        nsc = EPAD // sb
        return jnp.concatenate([srch.reshape(nsc, k, 128),
                                dsth.reshape(nsc, k, 128)], axis=1)

    sd128, sd256 = pack_idx(128), pack_idx(256)
    ones_e = jnp.ones((npad_e,), jnp.float32)
    ax = jnp.concatenate([edge_attr[:, 0], ones_e])
    ay = jnp.concatenate([edge_attr[:, 1], ones_e])
    az = jnp.concatenate([edge_attr[:, 2], ones_e])
    x2 = jnp.concatenate([x.astype(jnp.int32),
                          jnp.full((NPAD - N,), 8, jnp.int32)]).reshape(NPAD, 1)

    e0 = _tc_embed(x2, emb_table)
    w0, w1, w2 = _tc_sh(ax, ay, az)
    wcat = jnp.stack([w0, w1, w2], axis=0)
    wp128 = wcat.reshape(3, EPAD // 128, 128).transpose(1, 0, 2)
    wp256 = wcat.reshape(3, EPAD // 256, 256).transpose(1, 0, 2)

    pa, pb = _multi_fn(
        32, 128,
        ((1, (("copy", 0, 0), ("scale", 0, 0, 16)), False, 0),),
        ((1, (("scale", 1, 0, 0), ("scale", 2, 0, 16)), False, 1),),
        2, 1)(e0, e0, sd128, wp128)
    h0a, h0b, g0, h1a, h1b, h1c = _tc_layer1(pa, pb, W1_00, W1_01, W2_01)

    pc0, pc1 = _agg(h0a, sd128, wp128, 32, [], direct=True, sb=128)
    pe0, pe1 = _agg(g0, sd128, wp128, 32,
                    [("scale", 0, 0, 0), ("scale", 1, 0, 16)], sb=128)
    pgd, pf = _multi_fn(
        16, 256,
        ((3, (("dot", 0),), False, 0),),
        ((1, (("scale", 2, 0, 0),), False, 1),),
        2, 3)(h1a, h1b, h1c, g0, sd256, wp256)
    ph0, ph1 = _multi_fn(
        16, 256,
        ((1, (("copy", 0, 0),), False, 0),),
        ((1, (("copy", 0, 0),), False, 1),),
        2, 1)(h1a, h1b, sd256, wp256)
    ph2, pd = _multi_fn(
        16, 256,
        ((1, (("copy", 0, 0),), False, 0),),
        ((1, (), True, 1),),
        2, 1)(h1c, h0b, sd256, wp256)
    h0a2, h0b2, h1a2, h1b2, h1c2 = _tc_layer2(
        pc0, pc1, pd, pe0, pe1, pf, pgd, (ph0, ph1, ph2),
        W2_00, W2_11, W2_10)

    pi0, pi1 = _agg(h0a2, sd128, wp128, 32, [], direct=True, sb=128)
    pk, pj = _multi_fn(
        16, 256,
        ((3, (("dot", 0),), False, 0),),
        ((1, (), True, 1),),
        2, 3)(h1a2, h1b2, h1c2, h0b2, sd256, wp256)
    out = _tc_layer3(pi0, pi1, pj, pk, W3_00, W3_11, Hw1, Hw2)

    return out[:N]
